# baseline (speedup 1.0000x reference)
import functools

import jax
import jax.numpy as jnp
from jax.experimental import pallas as pl
from jax.experimental.pallas import tpu as pltpu

B_N = 9


def _body(batch, hw, nsteps, tile, fv_ref, mb_ref, pix_ref, img_ref,
          acc_ref, bank_ref, fv2_ref, mbn_ref):
    i = pl.program_id(0)
    fv = fv_ref[...]
    q, c = fv.shape

    @pl.when(i == 0)
    def _():
        fv2_ref[...] = fv * -2.0
        acc_ref[...] = jnp.full((q, c), jnp.inf, fv.dtype)

    @pl.when(i < nsteps)
    def _():
        mb = mb_ref[...]
        bank_ref[pl.ds(i * tile, tile), :] = mb
        prod2 = jax.lax.dot_general(fv2_ref[...], mb,
                                    (((1,), (1,)), ((), ())))
        mbn = jax.lax.dot_general(jnp.ones((1, c), fv.dtype), mb * mb,
                                  (((1,), (1,)), ((), ())))
        mbn_ref[i] = mbn
        tt = prod2 + mbn
        parts = [tt[:, k * c:(k + 1) * c] for k in range(tile // c)]
        while len(parts) > 1:
            parts = [jnp.minimum(parts[j], parts[j + 1])
                     for j in range(0, len(parts) - 1, 2)] + (
                         [parts[-1]] if len(parts) % 2 else [])
        part = parts[0]
        acc_ref[...] = jnp.minimum(acc_ref[...], part)

    @pl.when(i == nsteps)
    def _():
        big = jnp.int32(2 ** 30)
        fvn = jnp.sum(fv * fv, axis=1, keepdims=True)
        mnd = jnp.min(acc_ref[...], axis=1, keepdims=True) + fvn
        s = jnp.sqrt(jnp.maximum(mnd, 0.0))
        pix_ref[...] = s

        row_iota = jax.lax.broadcasted_iota(jnp.int32, (q, 1), 0)
        col_b = jax.lax.broadcasted_iota(jnp.int32, (q, batch), 1)
        in_b = (row_iota >= col_b * hw) & (row_iota < (col_b + 1) * hw)
        sb = jnp.where(in_b, s, -jnp.inf)
        mx = jnp.max(sb, axis=0, keepdims=True)
        idx = jnp.min(jnp.where(sb == mx, row_iota, big),
                      axis=0, keepdims=True)
        onehot = (row_iota == idx).astype(fv.dtype)
        sel = jax.lax.dot_general(onehot, fv, (((0,), (0,)), ((), ())))

        bank = bank_ref[...]
        prod2 = jax.lax.dot_general(sel * -2.0, bank,
                                    (((1,), (1,)), ((), ())))
        seln = jnp.sum(sel * sel, axis=1, keepdims=True)
        chunks = [mbn_ref[k] + prod2[:, k * tile:(k + 1) * tile]
                  for k in range(nsteps)]

        g = nsteps // 4
        r = [jnp.concatenate(chunks[k * g:(k + 1) * g], axis=1)
             for k in range(4)]
        for a, b in ((0, 1), (2, 3), (0, 2), (1, 3), (1, 2)):
            lo = jnp.minimum(r[a], r[b])
            hi = jnp.maximum(r[a], r[b])
            r[a], r[b] = lo, hi
        r0, r1, r2, r3 = r
        col_iota = jax.lax.broadcasted_iota(jnp.int32, r0.shape, 1)
        sds = []
        for _ in range(B_N):
            mn = jnp.min(r0, axis=1, keepdims=True)
            sds.append(jnp.sqrt(jnp.maximum(mn + seln, 0.0)))
            amn = jnp.min(jnp.where(r0 == mn, col_iota, big),
                          axis=1, keepdims=True)
            hit = col_iota == amn
            r0 = jnp.where(hit, r1, r0)
            r1 = jnp.where(hit, r2, r1)
            r2 = jnp.where(hit, r3, r2)
            r3 = jnp.where(hit, jnp.inf, r3)

        top = sds[-1]
        esum = jnp.zeros_like(top)
        for sd in sds:
            esum = esum + jnp.exp(sd - top)
        p0 = jnp.exp(sds[0] - top) / esum
        img_ref[...] = sds[0] * (1.0 - p0)


def kernel(feature_batch, mb):
    batch, height, width, channels = feature_batch.shape
    hw = height * width
    q = batch * hw
    m = mb.shape[0]
    fv = jnp.reshape(feature_batch, (q, channels))

    tile = 1024
    nsteps = m // tile
    pix, img = pl.pallas_call(
        functools.partial(_body, batch, hw, nsteps, tile),
        grid=(nsteps + 1,),
        in_specs=[
            pl.BlockSpec((q, channels), lambda i: (0, 0)),
            pl.BlockSpec((tile, channels),
                         lambda i: (jnp.minimum(i, nsteps - 1), 0)),
        ],
        out_specs=[
            pl.BlockSpec((q, 1), lambda i: (0, 0)),
            pl.BlockSpec((batch, 1), lambda i: (0, 0)),
        ],
        out_shape=[
            jax.ShapeDtypeStruct((q, 1), fv.dtype),
            jax.ShapeDtypeStruct((batch, 1), fv.dtype),
        ],
        scratch_shapes=[
            pltpu.VMEM((q, channels), fv.dtype),
            pltpu.VMEM((m, channels), fv.dtype),
            pltpu.VMEM((q, channels), fv.dtype),
            pltpu.VMEM((nsteps, 1, tile), fv.dtype),
        ],
    )(fv, mb)

    pixel_scores = jnp.reshape(pix, (batch, 1, height, width))
    image_scores = jnp.reshape(img, (batch,))
    return (pixel_scores, image_scores)


# speedup vs baseline: 1.1025x; 1.1025x over previous
import functools

import jax
import jax.numpy as jnp
from jax.experimental import pallas as pl
from jax.experimental.pallas import tpu as pltpu

B_N = 9


def _body(batch, hw, nsteps, tile, fv_ref, mb_ref, pix_ref, img_ref,
          acc_ref, bank_ref, fv2_ref, mbn_ref):
    i = pl.program_id(0)
    q, c = fv_ref.shape

    @pl.when(i == 0)
    def _():
        fv2_ref[...] = fv_ref[...] * -2.0
        acc_ref[...] = jnp.full((q, c), jnp.inf, fv_ref.dtype)

    @pl.when(i < nsteps)
    def _():
        mb = mb_ref[...]
        bank_ref[pl.ds(i * tile, tile), :] = mb
        prod2 = jax.lax.dot_general(fv2_ref[...], mb,
                                    (((1,), (1,)), ((), ())))
        mbn = jax.lax.dot_general(jnp.ones((1, c), mb.dtype), mb * mb,
                                  (((1,), (1,)), ((), ())))
        mbn_ref[i] = mbn
        tt = prod2 + mbn
        parts = [tt[:, k * c:(k + 1) * c] for k in range(tile // c)]
        while len(parts) > 1:
            parts = [jnp.minimum(parts[j], parts[j + 1])
                     for j in range(0, len(parts) - 1, 2)] + (
                         [parts[-1]] if len(parts) % 2 else [])
        part = parts[0]
        acc_ref[...] = jnp.minimum(acc_ref[...], part)

    @pl.when(i == nsteps)
    def _():
        big = jnp.int32(2 ** 30)
        fv = fv_ref[...]
        fvn = jnp.sum(fv * fv, axis=1, keepdims=True)
        mnd = jnp.min(acc_ref[...], axis=1, keepdims=True) + fvn
        s = jnp.sqrt(jnp.maximum(mnd, 0.0))
        pix_ref[...] = s

        row_iota = jax.lax.broadcasted_iota(jnp.int32, (q, 1), 0)
        col_b = jax.lax.broadcasted_iota(jnp.int32, (q, batch), 1)
        in_b = (row_iota >= col_b * hw) & (row_iota < (col_b + 1) * hw)
        sb = jnp.where(in_b, s, -jnp.inf)
        mx = jnp.max(sb, axis=0, keepdims=True)
        idx = jnp.min(jnp.where(sb == mx, row_iota, big),
                      axis=0, keepdims=True)
        onehot = (row_iota == idx).astype(fv.dtype)
        sel = jax.lax.dot_general(onehot, fv, (((0,), (0,)), ((), ())))

        bank = bank_ref[...]
        prod2 = jax.lax.dot_general(sel * -2.0, bank,
                                    (((1,), (1,)), ((), ())))
        seln = jnp.sum(sel * sel, axis=1, keepdims=True)
        chunks = [mbn_ref[k] + prod2[:, k * tile:(k + 1) * tile]
                  for k in range(nsteps)]

        g = nsteps // 4
        r = [jnp.concatenate(chunks[k * g:(k + 1) * g], axis=1)
             for k in range(4)]
        for a, b in ((0, 1), (2, 3), (0, 2), (1, 3), (1, 2)):
            lo = jnp.minimum(r[a], r[b])
            hi = jnp.maximum(r[a], r[b])
            r[a], r[b] = lo, hi
        r0, r1, r2, r3 = r
        col_iota = jax.lax.broadcasted_iota(jnp.int32, r0.shape, 1)
        sds = []
        for _ in range(B_N):
            mn = jnp.min(r0, axis=1, keepdims=True)
            sds.append(jnp.sqrt(jnp.maximum(mn + seln, 0.0)))
            amn = jnp.min(jnp.where(r0 == mn, col_iota, big),
                          axis=1, keepdims=True)
            hit = col_iota == amn
            r0 = jnp.where(hit, r1, r0)
            r1 = jnp.where(hit, r2, r1)
            r2 = jnp.where(hit, r3, r2)
            r3 = jnp.where(hit, jnp.inf, r3)

        top = sds[-1]
        esum = jnp.zeros_like(top)
        for sd in sds:
            esum = esum + jnp.exp(sd - top)
        p0 = jnp.exp(sds[0] - top) / esum
        img_ref[...] = sds[0] * (1.0 - p0)


def kernel(feature_batch, mb):
    batch, height, width, channels = feature_batch.shape
    hw = height * width
    q = batch * hw
    m = mb.shape[0]
    fv = jnp.reshape(feature_batch, (q, channels))

    tile = 2048
    nsteps = m // tile
    pix, img = pl.pallas_call(
        functools.partial(_body, batch, hw, nsteps, tile),
        grid=(nsteps + 1,),
        in_specs=[
            pl.BlockSpec((q, channels), lambda i: (0, 0)),
            pl.BlockSpec((tile, channels),
                         lambda i: (jnp.minimum(i, nsteps - 1), 0)),
        ],
        out_specs=[
            pl.BlockSpec((q, 1), lambda i: (0, 0)),
            pl.BlockSpec((batch, 1), lambda i: (0, 0)),
        ],
        out_shape=[
            jax.ShapeDtypeStruct((q, 1), fv.dtype),
            jax.ShapeDtypeStruct((batch, 1), fv.dtype),
        ],
        scratch_shapes=[
            pltpu.VMEM((q, channels), fv.dtype),
            pltpu.VMEM((m, channels), fv.dtype),
            pltpu.VMEM((q, channels), fv.dtype),
            pltpu.VMEM((nsteps, 1, tile), fv.dtype),
        ],
    )(fv, mb)

    pixel_scores = jnp.reshape(pix, (batch, 1, height, width))
    image_scores = jnp.reshape(img, (batch,))
    return (pixel_scores, image_scores)
